# baseline (device time: 76998 ns/iter reference)
import jax
import jax.numpy as jnp
from jax import lax
from jax.experimental import pallas as pl
from jax.experimental.pallas import tpu as pltpu

N_DEV = 16
E_LOCAL = 2


def kernel(x, router_W, route_idx, expert_W):
    n_tok, d_model = x.shape
    n_exp = router_W.shape[1]
    d_out = expert_W.shape[2]

    def body(x_ref, rw_ref, idx_ref, ew_ref, out_ref, comm_ref, send_sems, recv_sems):
        my_pos = lax.axis_index("i")
        left = lax.rem(my_pos - 1 + N_DEV, N_DEV)
        right = lax.rem(my_pos + 1, N_DEV)

        barrier_sem = pltpu.get_barrier_semaphore()
        for nbr in (left, right):
            pl.semaphore_signal(
                barrier_sem, inc=1,
                device_id=(nbr,), device_id_type=pl.DeviceIdType.MESH,
            )
        pl.semaphore_wait(barrier_sem, 2)

        xv = x_ref[:, :]
        scores = jnp.dot(xv, rw_ref[:, :], preferred_element_type=jnp.float32)
        m = jnp.max(scores, axis=-1, keepdims=True)
        e = jnp.exp(scores - m)
        probs = e / jnp.sum(e, axis=-1, keepdims=True)

        idx0 = idx_ref[:, 0:1]
        idx1 = idx_ref[:, 1:2]
        eids = lax.broadcasted_iota(jnp.int32, (n_tok, n_exp), 1)
        sel = jnp.logical_or(eids == idx0, eids == idx1)
        denom = jnp.sum(jnp.where(sel, probs, 0.0), axis=-1, keepdims=True)

        acc = jnp.zeros((n_tok, d_out), jnp.float32)
        for l in range(E_LOCAL):
            ge = my_pos * E_LOCAL + l
            pe = jnp.sum(
                jnp.where(eids == ge, probs, 0.0), axis=-1, keepdims=True
            )
            mask = jnp.logical_or(idx0 == ge, idx1 == ge)
            w = jnp.where(mask, pe / denom, 0.0)
            xw = (xv * w).astype(jnp.bfloat16)
            acc = acc + jnp.dot(
                xw, ew_ref[l].astype(jnp.bfloat16),
                preferred_element_type=jnp.float32,
            )

        out_ref[:, :] = acc
        comm_ref[0] = acc

        for h in range(N_DEV - 1):
            s = h % 2
            r = (h + 1) % 2
            rdma = pltpu.make_async_remote_copy(
                src_ref=comm_ref.at[s],
                dst_ref=comm_ref.at[r],
                send_sem=send_sems.at[s],
                recv_sem=recv_sems.at[r],
                device_id=(right,),
                device_id_type=pl.DeviceIdType.MESH,
            )
            rdma.start()
            rdma.wait()
            out_ref[:, :] = out_ref[:, :] + comm_ref[r]

    return pl.pallas_call(
        body,
        out_shape=jax.ShapeDtypeStruct((n_tok, d_out), jnp.float32),
        in_specs=[
            pl.BlockSpec(memory_space=pltpu.VMEM),
            pl.BlockSpec(memory_space=pltpu.VMEM),
            pl.BlockSpec(memory_space=pltpu.VMEM),
            pl.BlockSpec(memory_space=pltpu.VMEM),
        ],
        out_specs=pl.BlockSpec(memory_space=pltpu.VMEM),
        scratch_shapes=[
            pltpu.VMEM((2, n_tok, d_out), jnp.float32),
            pltpu.SemaphoreType.DMA((2,)),
            pltpu.SemaphoreType.DMA((2,)),
        ],
        compiler_params=pltpu.CompilerParams(collective_id=0),
    )(x, router_W, route_idx, expert_W)


# device time: 23212 ns/iter; 3.3172x vs baseline; 3.3172x over previous
import jax
import jax.numpy as jnp
from jax import lax
from jax.experimental import pallas as pl
from jax.experimental.pallas import tpu as pltpu

N_DEV = 16
E_LOCAL = 2
N_ROUNDS = 4


def kernel(x, router_W, route_idx, expert_W):
    n_tok, d_model = x.shape
    n_exp = router_W.shape[1]
    d_out = expert_W.shape[2]

    def body(x_ref, rw_ref, idx_ref, ew_ref, out_ref, send_ref, comm_ref,
             send_sems, recv_sems):
        my_pos = lax.axis_index("i")
        partners = [jnp.bitwise_xor(my_pos, 1 << r) for r in range(N_ROUNDS)]

        barrier_sem = pltpu.get_barrier_semaphore()
        for nbr in partners:
            pl.semaphore_signal(
                barrier_sem, inc=1,
                device_id=(nbr,), device_id_type=pl.DeviceIdType.MESH,
            )
        pl.semaphore_wait(barrier_sem, N_ROUNDS)

        xv = x_ref[:, :]
        scores = jnp.dot(xv, rw_ref[:, :], preferred_element_type=jnp.float32)
        m = jnp.max(scores, axis=-1, keepdims=True)
        e = jnp.exp(scores - m)
        probs = e / jnp.sum(e, axis=-1, keepdims=True)

        idx0 = idx_ref[:, 0:1]
        idx1 = idx_ref[:, 1:2]
        eids = lax.broadcasted_iota(jnp.int32, (n_tok, n_exp), 1)
        sel = jnp.logical_or(eids == idx0, eids == idx1)
        denom = jnp.sum(jnp.where(sel, probs, 0.0), axis=-1, keepdims=True)

        acc = jnp.zeros((n_tok, d_out), jnp.float32)
        for l in range(E_LOCAL):
            ge = my_pos * E_LOCAL + l
            pe = jnp.sum(
                jnp.where(eids == ge, probs, 0.0), axis=-1, keepdims=True
            )
            mask = jnp.logical_or(idx0 == ge, idx1 == ge)
            w = jnp.where(mask, pe / denom, 0.0)
            xw = (xv * w).astype(jnp.bfloat16)
            acc = acc + jnp.dot(
                xw, ew_ref[l].astype(jnp.bfloat16),
                preferred_element_type=jnp.float32,
            )

        for r in range(N_ROUNDS):
            send_ref[:, :] = acc.astype(jnp.bfloat16)
            rdma = pltpu.make_async_remote_copy(
                src_ref=send_ref,
                dst_ref=comm_ref.at[r],
                send_sem=send_sems.at[r],
                recv_sem=recv_sems.at[r],
                device_id=(partners[r],),
                device_id_type=pl.DeviceIdType.MESH,
            )
            rdma.start()
            rdma.wait()
            acc = acc + comm_ref[r].astype(jnp.float32)

        out_ref[:, :] = acc

    return pl.pallas_call(
        body,
        out_shape=jax.ShapeDtypeStruct((n_tok, d_out), jnp.float32),
        in_specs=[
            pl.BlockSpec(memory_space=pltpu.VMEM),
            pl.BlockSpec(memory_space=pltpu.VMEM),
            pl.BlockSpec(memory_space=pltpu.VMEM),
            pl.BlockSpec(memory_space=pltpu.VMEM),
        ],
        out_specs=pl.BlockSpec(memory_space=pltpu.VMEM),
        scratch_shapes=[
            pltpu.VMEM((n_tok, d_out), jnp.bfloat16),
            pltpu.VMEM((N_ROUNDS, n_tok, d_out), jnp.bfloat16),
            pltpu.SemaphoreType.DMA((N_ROUNDS,)),
            pltpu.SemaphoreType.DMA((N_ROUNDS,)),
        ],
        compiler_params=pltpu.CompilerParams(collective_id=0),
    )(x, router_W, route_idx, expert_W)


# device time: 21996 ns/iter; 3.5005x vs baseline; 1.0553x over previous
import jax
import jax.numpy as jnp
from jax import lax
from jax.experimental import pallas as pl
from jax.experimental.pallas import tpu as pltpu

N_DEV = 16
E_LOCAL = 2
N_ROUNDS = 4


def kernel(x, router_W, route_idx, expert_W):
    n_tok, d_model = x.shape
    n_exp = router_W.shape[1]
    d_out = expert_W.shape[2]

    def body(x_ref, rw_ref, idx_ref, ew_ref, out_ref, send_ref, comm_ref,
             send_sems, recv_sems):
        my_pos = lax.axis_index("i")
        partners = [jnp.bitwise_xor(my_pos, 1 << r) for r in range(N_ROUNDS)]

        barrier_sem = pltpu.get_barrier_semaphore()
        for nbr in partners:
            pl.semaphore_signal(
                barrier_sem, inc=1,
                device_id=(nbr,), device_id_type=pl.DeviceIdType.MESH,
            )

        xv = x_ref[:, :]
        scores = jnp.dot(xv, rw_ref[:, :], preferred_element_type=jnp.float32)
        m = jnp.max(scores, axis=-1, keepdims=True)
        e = jnp.exp(scores - m)
        probs = e / jnp.sum(e, axis=-1, keepdims=True)

        idx0 = idx_ref[:, 0:1]
        idx1 = idx_ref[:, 1:2]
        eids = lax.broadcasted_iota(jnp.int32, (n_tok, n_exp), 1)
        sel = jnp.logical_or(eids == idx0, eids == idx1)
        denom = jnp.sum(jnp.where(sel, probs, 0.0), axis=-1, keepdims=True)

        acc = jnp.zeros((n_tok, d_out), jnp.float32)
        for l in range(E_LOCAL):
            ge = my_pos * E_LOCAL + l
            pe = jnp.sum(
                jnp.where(eids == ge, probs, 0.0), axis=-1, keepdims=True
            )
            mask = jnp.logical_or(idx0 == ge, idx1 == ge)
            w = jnp.where(mask, pe / denom, 0.0)
            xw = (xv * w).astype(jnp.bfloat16)
            acc = acc + jnp.dot(
                xw, ew_ref[l].astype(jnp.bfloat16),
                preferred_element_type=jnp.float32,
            )

        pl.semaphore_wait(barrier_sem, N_ROUNDS)

        rdmas = []
        for r in range(N_ROUNDS):
            send_ref[r] = acc.astype(jnp.bfloat16)
            rdma = pltpu.make_async_remote_copy(
                src_ref=send_ref.at[r],
                dst_ref=comm_ref.at[r],
                send_sem=send_sems.at[r],
                recv_sem=recv_sems.at[r],
                device_id=(partners[r],),
                device_id_type=pl.DeviceIdType.MESH,
            )
            rdma.start()
            rdmas.append(rdma)
            rdma.wait_recv()
            acc = acc + comm_ref[r].astype(jnp.float32)

        out_ref[:, :] = acc
        for rdma in rdmas:
            rdma.wait_send()

    return pl.pallas_call(
        body,
        out_shape=jax.ShapeDtypeStruct((n_tok, d_out), jnp.float32),
        in_specs=[
            pl.BlockSpec(memory_space=pltpu.VMEM),
            pl.BlockSpec(memory_space=pltpu.VMEM),
            pl.BlockSpec(memory_space=pltpu.VMEM),
            pl.BlockSpec(memory_space=pltpu.VMEM),
        ],
        out_specs=pl.BlockSpec(memory_space=pltpu.VMEM),
        scratch_shapes=[
            pltpu.VMEM((N_ROUNDS, n_tok, d_out), jnp.bfloat16),
            pltpu.VMEM((N_ROUNDS, n_tok, d_out), jnp.bfloat16),
            pltpu.SemaphoreType.DMA((N_ROUNDS,)),
            pltpu.SemaphoreType.DMA((N_ROUNDS,)),
        ],
        compiler_params=pltpu.CompilerParams(collective_id=0),
    )(x, router_W, route_idx, expert_W)


# device time: 18554 ns/iter; 4.1499x vs baseline; 1.1855x over previous
import jax
import jax.numpy as jnp
from jax import lax
from jax.experimental import pallas as pl
from jax.experimental.pallas import tpu as pltpu

N_DEV = 16
E_LOCAL = 2
N_ROUNDS = 4
N_CHAINS = 4


def kernel(x, router_W, route_idx, expert_W):
    n_tok, d_model = x.shape
    n_exp = router_W.shape[1]
    d_out = expert_W.shape[2]

    def body(x_ref, rw_ref, idx_ref, ew_ref, out_ref, send_ref, comm_ref,
             send_sems, recv_sems):
        my_pos = lax.axis_index("i")
        partners = [jnp.bitwise_xor(my_pos, 1 << r) for r in range(N_ROUNDS)]

        barrier_sem = pltpu.get_barrier_semaphore()
        for nbr in partners:
            pl.semaphore_signal(
                barrier_sem, inc=1,
                device_id=(nbr,), device_id_type=pl.DeviceIdType.MESH,
            )

        xv = x_ref[:, :]
        scores = jnp.dot(xv, rw_ref[:, :], preferred_element_type=jnp.float32)
        m = jnp.max(scores, axis=-1, keepdims=True)
        e = jnp.exp(scores - m)
        probs = e / jnp.sum(e, axis=-1, keepdims=True)

        idx0 = idx_ref[:, 0:1]
        idx1 = idx_ref[:, 1:2]
        eids = lax.broadcasted_iota(jnp.int32, (n_tok, n_exp), 1)
        sel = jnp.logical_or(eids == idx0, eids == idx1)
        denom = jnp.sum(jnp.where(sel, probs, 0.0), axis=-1, keepdims=True)

        acc = jnp.zeros((n_tok, d_out), jnp.float32)
        for l in range(E_LOCAL):
            ge = my_pos * E_LOCAL + l
            pe = jnp.sum(
                jnp.where(eids == ge, probs, 0.0), axis=-1, keepdims=True
            )
            mask = jnp.logical_or(idx0 == ge, idx1 == ge)
            w = jnp.where(mask, pe / denom, 0.0)
            xw = (xv * w).astype(jnp.bfloat16)
            acc = acc + jnp.dot(
                xw, ew_ref[l].astype(jnp.bfloat16),
                preferred_element_type=jnp.float32,
            )

        pl.semaphore_wait(barrier_sem, N_ROUNDS)

        rows = n_tok // N_CHAINS
        accs = [acc[c * rows:(c + 1) * rows, :] for c in range(N_CHAINS)]
        rdmas = [[None] * N_ROUNDS for _ in range(N_CHAINS)]
        for r in range(N_ROUNDS):
            for c in range(N_CHAINS):
                if r > 0:
                    rdmas[c][r - 1].wait_recv()
                    accs[c] = accs[c] + comm_ref[c, r - 1].astype(jnp.float32)
                send_ref[c, r] = accs[c].astype(jnp.bfloat16)
                rdma = pltpu.make_async_remote_copy(
                    src_ref=send_ref.at[c, r],
                    dst_ref=comm_ref.at[c, r],
                    send_sem=send_sems.at[c, r],
                    recv_sem=recv_sems.at[c, r],
                    device_id=(partners[(r + c) % N_ROUNDS],),
                    device_id_type=pl.DeviceIdType.MESH,
                )
                rdma.start()
                rdmas[c][r] = rdma
        for c in range(N_CHAINS):
            rdmas[c][N_ROUNDS - 1].wait_recv()
            accs[c] = accs[c] + comm_ref[c, N_ROUNDS - 1].astype(jnp.float32)
            out_ref[c * rows:(c + 1) * rows, :] = accs[c]
        for chain in rdmas:
            for rdma in chain:
                rdma.wait_send()

    return pl.pallas_call(
        body,
        out_shape=jax.ShapeDtypeStruct((n_tok, d_out), jnp.float32),
        in_specs=[
            pl.BlockSpec(memory_space=pltpu.VMEM),
            pl.BlockSpec(memory_space=pltpu.VMEM),
            pl.BlockSpec(memory_space=pltpu.VMEM),
            pl.BlockSpec(memory_space=pltpu.VMEM),
        ],
        out_specs=pl.BlockSpec(memory_space=pltpu.VMEM),
        scratch_shapes=[
            pltpu.VMEM(
                (N_CHAINS, N_ROUNDS, n_tok // N_CHAINS, d_out), jnp.bfloat16
            ),
            pltpu.VMEM(
                (N_CHAINS, N_ROUNDS, n_tok // N_CHAINS, d_out), jnp.bfloat16
            ),
            pltpu.SemaphoreType.DMA((N_CHAINS, N_ROUNDS)),
            pltpu.SemaphoreType.DMA((N_CHAINS, N_ROUNDS)),
        ],
        compiler_params=pltpu.CompilerParams(collective_id=0),
    )(x, router_W, route_idx, expert_W)


# device time: 15606 ns/iter; 4.9339x vs baseline; 1.1889x over previous
import jax
import jax.numpy as jnp
from jax import lax
from jax.experimental import pallas as pl
from jax.experimental.pallas import tpu as pltpu

N_DEV = 16
E_LOCAL = 2
N_ROUNDS = 4
N_CHAINS = 4
PLANE_MASKS = (1, 3, 2)
ZCOL_MASKS = (4, 8, 12)
N_PHASES = 2
RADIX = 3


def kernel(x, router_W, route_idx, expert_W):
    n_tok, d_model = x.shape
    n_exp = router_W.shape[1]
    d_out = expert_W.shape[2]

    def body(x_ref, rw_ref, idx_ref, ew_ref, out_ref, send_ref, comm_ref,
             send_sems, recv_sems):
        my_pos = lax.axis_index("i")
        all_masks = PLANE_MASKS + ZCOL_MASKS
        partner = {m: jnp.bitwise_xor(my_pos, m) for m in all_masks}

        barrier_sem = pltpu.get_barrier_semaphore()
        for m in all_masks:
            pl.semaphore_signal(
                barrier_sem, inc=1,
                device_id=(partner[m],), device_id_type=pl.DeviceIdType.MESH,
            )

        xv = x_ref[:, :]
        scores = jnp.dot(xv, rw_ref[:, :], preferred_element_type=jnp.float32)

        idx0 = idx_ref[:, 0:1]
        idx1 = idx_ref[:, 1:2]
        eids = lax.broadcasted_iota(jnp.int32, (n_tok, n_exp), 1)
        s0 = jnp.sum(
            jnp.where(eids == idx0, scores, 0.0), axis=-1, keepdims=True
        )
        s1 = jnp.sum(
            jnp.where(eids == idx1, scores, 0.0), axis=-1, keepdims=True
        )
        m2 = jnp.maximum(s0, s1)
        e0 = jnp.exp(s0 - m2)
        e1 = jnp.exp(s1 - m2)
        denom = e0 + e1

        ws = []
        for l in range(E_LOCAL):
            ge = my_pos * E_LOCAL + l
            w = jnp.where(idx0 == ge, e0 / denom, 0.0)
            w = jnp.where(idx1 == ge, e1 / denom, w)
            ws.append(w)
        ew_b = [ew_ref[l].astype(jnp.bfloat16) for l in range(E_LOCAL)]

        pl.semaphore_wait(barrier_sem, len(all_masks))

        rows = n_tok // N_CHAINS
        chain_masks = [
            (PLANE_MASKS, ZCOL_MASKS) if c % 2 == 0 else (ZCOL_MASKS, PLANE_MASKS)
            for c in range(N_CHAINS)
        ]

        def start_sends(c, ph, acc_c):
            send_ref[c, ph] = acc_c.astype(jnp.bfloat16)
            out = []
            for j, m in enumerate(chain_masks[c][ph]):
                rdma = pltpu.make_async_remote_copy(
                    src_ref=send_ref.at[c, ph],
                    dst_ref=comm_ref.at[c, ph, j],
                    send_sem=send_sems.at[c, ph, j],
                    recv_sem=recv_sems.at[c, ph, j],
                    device_id=(partner[m],),
                    device_id_type=pl.DeviceIdType.MESH,
                )
                rdma.start()
                out.append(rdma)
            return out

        def wait_and_add(c, ph, acc_c):
            for j in range(RADIX):
                rdmas[c][ph][j].wait_recv()
                acc_c = acc_c + comm_ref[c, ph, j].astype(jnp.float32)
            return acc_c

        accs = []
        rdmas = [[None] * N_PHASES for _ in range(N_CHAINS)]
        for c in range(N_CHAINS):
            sl = slice(c * rows, (c + 1) * rows)
            acc_c = jnp.zeros((rows, d_out), jnp.float32)
            for l in range(E_LOCAL):
                xw = (xv[sl, :] * ws[l][sl, :]).astype(jnp.bfloat16)
                acc_c = acc_c + jnp.dot(
                    xw, ew_b[l], preferred_element_type=jnp.float32
                )
            accs.append(acc_c)
            rdmas[c][0] = start_sends(c, 0, acc_c)

        for c in (0, 2, 1, 3):
            accs[c] = wait_and_add(c, 0, accs[c])
            rdmas[c][1] = start_sends(c, 1, accs[c])
        for c in (1, 3, 0, 2):
            accs[c] = wait_and_add(c, 1, accs[c])
            out_ref[c * rows:(c + 1) * rows, :] = accs[c]
        for chain in rdmas:
            for phase in chain:
                for rdma in phase:
                    rdma.wait_send()

    return pl.pallas_call(
        body,
        out_shape=jax.ShapeDtypeStruct((n_tok, d_out), jnp.float32),
        in_specs=[
            pl.BlockSpec(memory_space=pltpu.VMEM),
            pl.BlockSpec(memory_space=pltpu.VMEM),
            pl.BlockSpec(memory_space=pltpu.VMEM),
            pl.BlockSpec(memory_space=pltpu.VMEM),
        ],
        out_specs=pl.BlockSpec(memory_space=pltpu.VMEM),
        scratch_shapes=[
            pltpu.VMEM(
                (N_CHAINS, N_PHASES, n_tok // N_CHAINS, d_out), jnp.bfloat16
            ),
            pltpu.VMEM(
                (N_CHAINS, N_PHASES, RADIX, n_tok // N_CHAINS, d_out),
                jnp.bfloat16,
            ),
            pltpu.SemaphoreType.DMA((N_CHAINS, N_PHASES, RADIX)),
            pltpu.SemaphoreType.DMA((N_CHAINS, N_PHASES, RADIX)),
        ],
        compiler_params=pltpu.CompilerParams(collective_id=0),
    )(x, router_W, route_idx, expert_W)
